# baseline (speedup 1.0000x reference)
import functools

import jax
import jax.numpy as jnp
from jax import lax
from jax.experimental import pallas as pl
from jax.experimental.pallas import tpu as pltpu
from jax.experimental.pallas import tpu_sc as plsc

NC = 2
NS = 16
NW = NC * NS
LANES = 16
C = 128
G = 8

_MESH = plsc.VectorSubcoreMesh(core_axis_name="c", subcore_axis_name="s")


def _zero_rows(buf, width):
  zero16 = jnp.zeros((LANES,), jnp.float32)

  def body(i, carry):
    for w0 in range(width // LANES):
      buf[i, pl.ds(w0 * LANES, LANES)] = zero16
    return carry

  lax.fori_loop(0, C, body, 0)


def _worker_range(cid, sid, k0, k1):
  my_k = jnp.where(cid == 0, k0, k1)
  start = jnp.where(cid == 0, sid * k0, NS * k0 + sid * k1)
  return start, my_k


def _make_deg_kernel(n_pad, k0, k1):
  rows_per_sub = n_pad // NS
  nz = rows_per_sub // C
  kmax = max(k0, k1)

  @functools.partial(
      pl.kernel,
      out_type=jax.ShapeDtypeStruct((NC, n_pad), jnp.float32),
      mesh=_MESH,
      compiler_params=pltpu.CompilerParams(use_tc_tiling_on_sc=False),
      scratch_types=[
          pltpu.VMEM((kmax, C), jnp.int32),
          pltpu.VMEM((C,), jnp.float32),
          pltpu.VMEM((rows_per_sub,), jnp.float32),
          pltpu.VMEM_SHARED((n_pad,), jnp.float32),
          pltpu.SemaphoreType.DMA,
          pltpu.SemaphoreType.DMA,
      ],
  )
  def deg_kernel(row_hbm, out_hbm, idx_v, ones_v, zbuf_v, acc_s, sem, sem_z):
    cid = lax.axis_index("c")
    sid = lax.axis_index("s")
    start, _ = _worker_range(cid, sid, k0, k1)
    ngroups = jnp.where(cid == 0, k0 // G, k1 // G)
    base = sid * rows_per_sub

    @pl.when(cid == 0)
    def _():
      pltpu.async_copy(row_hbm.at[pl.ds(start, k0)],
                       idx_v.at[pl.ds(0, k0)], sem)

    @pl.when(cid != 0)
    def _():
      pltpu.async_copy(row_hbm.at[pl.ds(start, k1)],
                       idx_v.at[pl.ds(0, k1)], sem)

    one16 = jnp.full((LANES,), 1.0, jnp.float32)
    zero16 = jnp.zeros((LANES,), jnp.float32)

    def fill(i, carry):
      zbuf_v[pl.ds(i * LANES, LANES)] = zero16
      return carry

    def fill1(i, carry):
      ones_v[pl.ds(i * LANES, LANES)] = one16
      return carry

    lax.fori_loop(0, rows_per_sub // LANES, fill, 0)
    lax.fori_loop(0, C // LANES, fill1, 0)

    pltpu.async_copy(zbuf_v, acc_s.at[pl.ds(base, rows_per_sub)], sem_z)
    pltpu.make_async_copy(
        zbuf_v, acc_s.at[pl.ds(base, rows_per_sub)], sem_z).wait()

    @pl.when(cid == 0)
    def _():
      pltpu.make_async_copy(row_hbm.at[pl.ds(start, k0)],
                            idx_v.at[pl.ds(0, k0)], sem).wait()

    @pl.when(cid != 0)
    def _():
      pltpu.make_async_copy(row_hbm.at[pl.ds(start, k1)],
                            idx_v.at[pl.ds(0, k1)], sem).wait()
    plsc.subcore_barrier()

    def fire(g):
      for u in range(G):
        pltpu.async_copy(ones_v, acc_s.at[idx_v.at[g * G + u]], sem, add=True)

    def drain():
      for _ in range(G):
        pltpu.make_async_copy(ones_v, acc_s.at[idx_v.at[0]], sem).wait()

    fire(0)

    def group(g, carry):
      drain()

      @pl.when(g + 1 < ngroups)
      def _():
        fire(g + 1)
      return carry

    lax.fori_loop(0, ngroups, group, 0)
    plsc.subcore_barrier()

    sl = pl.ds(base, rows_per_sub)
    pltpu.sync_copy(acc_s.at[sl], out_hbm.at[cid, sl])

  return deg_kernel


def _make_spmm_kernel(n_pad, k0, k1, width):
  rows_per_sub = n_pad // NS
  nz = rows_per_sub // C
  kmax = max(k0, k1)

  @functools.partial(
      pl.kernel,
      out_type=jax.ShapeDtypeStruct((NC, n_pad, width), jnp.float32),
      mesh=_MESH,
      compiler_params=pltpu.CompilerParams(use_tc_tiling_on_sc=False),
      scratch_types=[
          pltpu.VMEM((kmax, C), jnp.int32),
          pltpu.VMEM((kmax, C), jnp.int32),
          pltpu.VMEM((2 * G * C, width), jnp.float32),
          pltpu.VMEM_SHARED((n_pad, width), jnp.float32),
          pltpu.SemaphoreType.DMA,
          pltpu.SemaphoreType.DMA,
          pltpu.SemaphoreType.DMA,
      ],
  )
  def spmm_kernel(col_hbm, row_hbm, table_hbm, out_hbm,
                  colv, rowv, gbuf, acc_s, sem_g, sem_s, sem_z):
    cid = lax.axis_index("c")
    sid = lax.axis_index("s")
    start, _ = _worker_range(cid, sid, k0, k1)
    ngroups = jnp.where(cid == 0, k0 // G, k1 // G)
    base = sid * rows_per_sub

    @pl.when(cid == 0)
    def _():
      pltpu.async_copy(col_hbm.at[pl.ds(start, k0)],
                       colv.at[pl.ds(0, k0)], sem_g)
      pltpu.async_copy(row_hbm.at[pl.ds(start, k0)],
                       rowv.at[pl.ds(0, k0)], sem_s)

    @pl.when(cid != 0)
    def _():
      pltpu.async_copy(col_hbm.at[pl.ds(start, k1)],
                       colv.at[pl.ds(0, k1)], sem_g)
      pltpu.async_copy(row_hbm.at[pl.ds(start, k1)],
                       rowv.at[pl.ds(0, k1)], sem_s)

    zero16 = jnp.zeros((LANES,), jnp.float32)

    def zfill(i, carry):
      for w0 in range(width // LANES):
        gbuf[i, pl.ds(w0 * LANES, LANES)] = zero16
      return carry

    lax.fori_loop(0, rows_per_sub, zfill, 0)
    pltpu.async_copy(gbuf.at[pl.ds(0, rows_per_sub)],
                     acc_s.at[pl.ds(base, rows_per_sub), :], sem_z)
    pltpu.make_async_copy(gbuf.at[pl.ds(0, rows_per_sub)],
                          acc_s.at[pl.ds(base, rows_per_sub), :], sem_z).wait()

    @pl.when(cid == 0)
    def _():
      pltpu.make_async_copy(col_hbm.at[pl.ds(start, k0)],
                            colv.at[pl.ds(0, k0)], sem_g).wait()
      pltpu.make_async_copy(row_hbm.at[pl.ds(start, k0)],
                            rowv.at[pl.ds(0, k0)], sem_s).wait()

    @pl.when(cid != 0)
    def _():
      pltpu.make_async_copy(col_hbm.at[pl.ds(start, k1)],
                            colv.at[pl.ds(0, k1)], sem_g).wait()
      pltpu.make_async_copy(row_hbm.at[pl.ds(start, k1)],
                            rowv.at[pl.ds(0, k1)], sem_s).wait()
    plsc.subcore_barrier()

    def buf(slot, u):
      return gbuf.at[pl.ds((slot * G + u) * C, C), :]

    def fire_gathers(g, slot):
      for u in range(G):
        pltpu.async_copy(table_hbm.at[colv.at[g * G + u]], buf(slot, u),
                         sem_g)

    def drain(sem, dst_slot):
      for u in range(G):
        pltpu.make_async_copy(table_hbm.at[colv.at[0]], buf(dst_slot, u),
                              sem).wait()

    fire_gathers(0, 0)

    def group(g, carry):
      cur = lax.rem(g, 2)
      nxt = 1 - cur

      @pl.when(g >= 1)
      def _():
        drain(sem_s, nxt)
      drain(sem_g, cur)

      for u in range(G):
        pltpu.async_copy(buf(cur, u), acc_s.at[rowv.at[g * G + u]],
                         sem_s, add=True)

      @pl.when(g + 1 < ngroups)
      def _():
        fire_gathers(g + 1, nxt)
      return carry

    lax.fori_loop(0, ngroups, group, 0)
    drain(sem_s, lax.rem(ngroups - 1, 2))
    plsc.subcore_barrier()

    sl = pl.ds(base, rows_per_sub)
    pltpu.sync_copy(acc_s.at[sl, :], out_hbm.at[cid, sl, :])

  return spmm_kernel


def kernel(x, edge_index, W1, b1, Wc, bc):
  n, in_ch = x.shape
  hid = W1.shape[1]
  out_ch = Wc.shape[1]
  e = edge_index.shape[1]
  mu, p_exp, k_iters = 0.1, 2.0, 2
  lam = 2.0 * mu / p_exp
  del p_exp

  total_chunks = -(-e // C)
  pair = -(-total_chunks // NS)

  def split(r):
    kk0 = int(round(pair * r / (1.0 + r) / G)) * G
    kk0 = max(G, min(kk0, (pair // G) * G))
    kk1 = -(-(pair - kk0) // G) * G
    return kk0, kk1

  k0s, k1s = split(1.0)
  k0d, k1d = split(1.0)
  slots = NS * max(k0s + k1s, k0d + k1d)
  e_pad = slots * C
  n_pad = -(-n // (NS * C)) * (NS * C)
  if n_pad == n:
    n_pad += NS * C

  row = edge_index[0]
  col = edge_index[1]
  pad = e_pad - e
  colp = jnp.concatenate([col, jnp.zeros((pad,), jnp.int32)])
  rowp = jnp.concatenate([row, jnp.full((pad,), n, jnp.int32)])
  colm = colp.reshape(slots, C)
  rowm = rowp.reshape(slots, C)

  deg2 = _make_deg_kernel(n_pad, k0d, k1d)(rowm)

  def tc_pre(x_ref, w1_ref, b1_ref, wc_ref, bc_ref, deg_ref,
             x0_ref, taug_ref, dis_ref, dgi_ref):
    h = jnp.maximum(
        jnp.dot(x_ref[...], w1_ref[...],
                preferred_element_type=jnp.float32) + b1_ref[...], 0.0)
    x0 = jnp.dot(h, wc_ref[...], preferred_element_type=jnp.float32) \
        + bc_ref[...]
    dg = deg_ref[0, :n, :] + deg_ref[1, :n, :] + 1.0
    dis = jnp.broadcast_to(lax.rsqrt(dg), (n, out_ch))
    x0_ref[...] = x0
    dis_ref[...] = dis
    dgi_ref[...] = jnp.broadcast_to(1.0 / dg, (n, out_ch))
    taug_ref[:, 0:hid] = dis * x0
    taug_ref[:, hid:2 * hid] = dis

  x0, t0aug, dis_b, dgi_b = pl.pallas_call(
      tc_pre,
      out_shape=[
          jax.ShapeDtypeStruct((n, out_ch), jnp.float32),
          jax.ShapeDtypeStruct((n, 2 * out_ch), jnp.float32),
          jax.ShapeDtypeStruct((n, out_ch), jnp.float32),
          jax.ShapeDtypeStruct((n, out_ch), jnp.float32),
      ],
  )(x, W1, b1.reshape(1, hid), Wc, bc.reshape(1, out_ch),
    deg2.reshape(NC, n_pad, 1))

  accA = _make_spmm_kernel(n_pad, k0s, k1s, 2 * out_ch)(colm, rowm, t0aug)

  def tc_mid(acc_ref, x0_ref, dis_ref, dgi_ref,
             out1_ref, t1_ref, alpha_ref):
    dis = dis_ref[...]
    dgi = dgi_ref[...]
    x0v = x0_ref[...]
    s = acc_ref[0, :n, out_ch:2 * out_ch] + acc_ref[1, :n, out_ch:2 * out_ch]
    denom = dis * s + dgi + lam
    alpha = 1.0 / denom
    beta = lam * alpha
    agg = dis * (acc_ref[0, :n, 0:out_ch] + acc_ref[1, :n, 0:out_ch]) \
        + dgi * x0v
    out1 = alpha * agg + beta * x0v
    out1_ref[...] = out1
    t1_ref[...] = dis * out1
    alpha_ref[...] = alpha

  out1, t1, alpha_b = pl.pallas_call(
      tc_mid,
      out_shape=[
          jax.ShapeDtypeStruct((n, out_ch), jnp.float32),
          jax.ShapeDtypeStruct((n, out_ch), jnp.float32),
          jax.ShapeDtypeStruct((n, out_ch), jnp.float32),
      ],
  )(accA, x0, dis_b, dgi_b)

  accB = _make_spmm_kernel(n_pad, k0s, k1s, out_ch)(colm, rowm, t1)

  def tc_fin(acc_ref, out1_ref, alpha_ref, x0_ref, dis_ref, dgi_ref, res_ref):
    dis = dis_ref[...]
    alpha = alpha_ref[...]
    agg = dis * (acc_ref[0, :n, :] + acc_ref[1, :n, :]) \
        + dgi_ref[...] * out1_ref[...]
    out2 = alpha * agg + (lam * alpha) * x0_ref[...]
    m = jnp.max(out2, axis=1, keepdims=True)
    lse = m + jnp.log(jnp.sum(jnp.exp(out2 - m), axis=1, keepdims=True))
    res_ref[...] = out2 - lse

  res = pl.pallas_call(
      tc_fin,
      out_shape=jax.ShapeDtypeStruct((n, out_ch), jnp.float32),
  )(accB, out1, alpha_b, x0, dis_b, dgi_b)

  del k_iters
  return res


# speedup vs baseline: 1.5249x; 1.5249x over previous
import functools

import jax
import jax.numpy as jnp
from jax import lax
from jax.experimental import pallas as pl
from jax.experimental.pallas import tpu as pltpu
from jax.experimental.pallas import tpu_sc as plsc

NC = 2
NS = 16
NW = NC * NS
LANES = 16
C = 128
G = 8

_MESH = plsc.VectorSubcoreMesh(core_axis_name="c", subcore_axis_name="s")


def _zero_rows(buf, width):
  zero16 = jnp.zeros((LANES,), jnp.float32)

  def body(i, carry):
    for w0 in range(width // LANES):
      buf[i, pl.ds(w0 * LANES, LANES)] = zero16
    return carry

  lax.fori_loop(0, C, body, 0)


def _worker_range(cid, sid, k0, k1):
  my_k = jnp.where(cid == 0, k0, k1)
  start = jnp.where(cid == 0, sid * k0, NS * k0 + sid * k1)
  return start, my_k


def _make_deg_kernel(n_pad, k0, k1):
  rows_per_sub = n_pad // NS
  nz = rows_per_sub // C
  kmax = max(k0, k1)

  @functools.partial(
      pl.kernel,
      out_type=jax.ShapeDtypeStruct((NC, n_pad), jnp.float32),
      mesh=_MESH,
      compiler_params=pltpu.CompilerParams(use_tc_tiling_on_sc=False),
      scratch_types=[
          pltpu.VMEM((kmax, C), jnp.int32),
          pltpu.VMEM((C,), jnp.float32),
          pltpu.VMEM((rows_per_sub,), jnp.float32),
          pltpu.VMEM_SHARED((n_pad,), jnp.float32),
          pltpu.SemaphoreType.DMA,
          pltpu.SemaphoreType.DMA,
      ],
  )
  def deg_kernel(row_hbm, out_hbm, idx_v, ones_v, zbuf_v, acc_s, sem, sem_z):
    cid = lax.axis_index("c")
    sid = lax.axis_index("s")
    start, _ = _worker_range(cid, sid, k0, k1)
    ngroups = jnp.where(cid == 0, k0 // G, k1 // G)
    base = sid * rows_per_sub

    @pl.when(cid == 0)
    def _():
      pltpu.async_copy(row_hbm.at[pl.ds(start, k0)],
                       idx_v.at[pl.ds(0, k0)], sem)

    @pl.when(cid != 0)
    def _():
      pltpu.async_copy(row_hbm.at[pl.ds(start, k1)],
                       idx_v.at[pl.ds(0, k1)], sem)

    one16 = jnp.full((LANES,), 1.0, jnp.float32)
    zero16 = jnp.zeros((LANES,), jnp.float32)

    def fill(i, carry):
      zbuf_v[pl.ds(i * LANES, LANES)] = zero16
      return carry

    def fill1(i, carry):
      ones_v[pl.ds(i * LANES, LANES)] = one16
      return carry

    lax.fori_loop(0, rows_per_sub // LANES, fill, 0)
    lax.fori_loop(0, C // LANES, fill1, 0)

    pltpu.async_copy(zbuf_v, acc_s.at[pl.ds(base, rows_per_sub)], sem_z)
    pltpu.make_async_copy(
        zbuf_v, acc_s.at[pl.ds(base, rows_per_sub)], sem_z).wait()

    @pl.when(cid == 0)
    def _():
      pltpu.make_async_copy(row_hbm.at[pl.ds(start, k0)],
                            idx_v.at[pl.ds(0, k0)], sem).wait()

    @pl.when(cid != 0)
    def _():
      pltpu.make_async_copy(row_hbm.at[pl.ds(start, k1)],
                            idx_v.at[pl.ds(0, k1)], sem).wait()
    plsc.subcore_barrier()

    def fire(g):
      for u in range(G):
        pltpu.async_copy(ones_v, acc_s.at[idx_v.at[g * G + u]], sem, add=True)

    def drain():
      for _ in range(G):
        pltpu.make_async_copy(ones_v, acc_s.at[idx_v.at[0]], sem).wait()

    fire(0)

    def group(g, carry):
      drain()

      @pl.when(g + 1 < ngroups)
      def _():
        fire(g + 1)
      return carry

    lax.fori_loop(0, ngroups, group, 0)
    plsc.subcore_barrier()

    sl = pl.ds(base, rows_per_sub)
    pltpu.sync_copy(acc_s.at[sl], out_hbm.at[cid, sl])

  return deg_kernel


def _make_spmm_kernel(n, n_pad, k0, k1, width):
  rows_per_sub = n_pad // NS
  tab_per_sub = n // NS
  kmax = max(k0, k1)

  @functools.partial(
      pl.kernel,
      out_type=jax.ShapeDtypeStruct((NC, n_pad, width), jnp.float32),
      mesh=_MESH,
      compiler_params=pltpu.CompilerParams(use_tc_tiling_on_sc=False),
      scratch_types=[
          pltpu.VMEM((kmax, C), jnp.int32),
          pltpu.VMEM((kmax, C), jnp.int32),
          pltpu.VMEM((2 * G * C, width), jnp.float32),
          pltpu.VMEM_SHARED((n_pad, width), jnp.float32),
          pltpu.VMEM_SHARED((n, width), jnp.float32),
          pltpu.SemaphoreType.DMA,
          pltpu.SemaphoreType.DMA,
          pltpu.SemaphoreType.DMA,
      ],
  )
  def spmm_kernel(col_hbm, row_hbm, table_hbm, out_hbm,
                  colv, rowv, gbuf, acc_s, tab_s, sem_g, sem_s, sem_z):
    cid = lax.axis_index("c")
    sid = lax.axis_index("s")
    start, _ = _worker_range(cid, sid, k0, k1)
    ngroups = jnp.where(cid == 0, k0 // G, k1 // G)
    base = sid * rows_per_sub

    tsl = pl.ds(sid * tab_per_sub, tab_per_sub)
    pltpu.async_copy(table_hbm.at[tsl, :], tab_s.at[tsl, :], sem_z)

    @pl.when(cid == 0)
    def _():
      pltpu.async_copy(col_hbm.at[pl.ds(start, k0)],
                       colv.at[pl.ds(0, k0)], sem_g)
      pltpu.async_copy(row_hbm.at[pl.ds(start, k0)],
                       rowv.at[pl.ds(0, k0)], sem_s)

    @pl.when(cid != 0)
    def _():
      pltpu.async_copy(col_hbm.at[pl.ds(start, k1)],
                       colv.at[pl.ds(0, k1)], sem_g)
      pltpu.async_copy(row_hbm.at[pl.ds(start, k1)],
                       rowv.at[pl.ds(0, k1)], sem_s)

    zero16 = jnp.zeros((LANES,), jnp.float32)

    def zfill(i, carry):
      for w0 in range(width // LANES):
        gbuf[i, pl.ds(w0 * LANES, LANES)] = zero16
      return carry

    lax.fori_loop(0, rows_per_sub, zfill, 0)
    pltpu.async_copy(gbuf.at[pl.ds(0, rows_per_sub)],
                     acc_s.at[pl.ds(base, rows_per_sub), :], sem_z)
    pltpu.make_async_copy(gbuf.at[pl.ds(0, rows_per_sub)],
                          acc_s.at[pl.ds(base, rows_per_sub), :], sem_z).wait()
    pltpu.make_async_copy(table_hbm.at[tsl, :], tab_s.at[tsl, :], sem_z).wait()

    @pl.when(cid == 0)
    def _():
      pltpu.make_async_copy(col_hbm.at[pl.ds(start, k0)],
                            colv.at[pl.ds(0, k0)], sem_g).wait()
      pltpu.make_async_copy(row_hbm.at[pl.ds(start, k0)],
                            rowv.at[pl.ds(0, k0)], sem_s).wait()

    @pl.when(cid != 0)
    def _():
      pltpu.make_async_copy(col_hbm.at[pl.ds(start, k1)],
                            colv.at[pl.ds(0, k1)], sem_g).wait()
      pltpu.make_async_copy(row_hbm.at[pl.ds(start, k1)],
                            rowv.at[pl.ds(0, k1)], sem_s).wait()
    plsc.subcore_barrier()

    def buf(slot, u):
      return gbuf.at[pl.ds((slot * G + u) * C, C), :]

    def fire_gathers(g, slot):
      for u in range(G):
        pltpu.async_copy(tab_s.at[colv.at[g * G + u]], buf(slot, u),
                         sem_g)

    def drain(sem, dst_slot):
      for u in range(G):
        pltpu.make_async_copy(tab_s.at[colv.at[0]], buf(dst_slot, u),
                              sem).wait()

    fire_gathers(0, 0)

    def group(g, carry):
      cur = lax.rem(g, 2)
      nxt = 1 - cur

      @pl.when(g >= 1)
      def _():
        drain(sem_s, nxt)
      drain(sem_g, cur)

      for u in range(G):
        pltpu.async_copy(buf(cur, u), acc_s.at[rowv.at[g * G + u]],
                         sem_s, add=True)

      @pl.when(g + 1 < ngroups)
      def _():
        fire_gathers(g + 1, nxt)
      return carry

    lax.fori_loop(0, ngroups, group, 0)
    drain(sem_s, lax.rem(ngroups - 1, 2))
    plsc.subcore_barrier()

    sl = pl.ds(base, rows_per_sub)
    pltpu.sync_copy(acc_s.at[sl, :], out_hbm.at[cid, sl, :])

  return spmm_kernel


def kernel(x, edge_index, W1, b1, Wc, bc):
  n, in_ch = x.shape
  hid = W1.shape[1]
  out_ch = Wc.shape[1]
  e = edge_index.shape[1]
  mu, p_exp, k_iters = 0.1, 2.0, 2
  lam = 2.0 * mu / p_exp
  del p_exp

  total_chunks = -(-e // C)
  pair = -(-total_chunks // NS)

  def split(r):
    kk0 = int(round(pair * r / (1.0 + r) / G)) * G
    kk0 = max(G, min(kk0, (pair // G) * G))
    kk1 = -(-(pair - kk0) // G) * G
    return kk0, kk1

  k0s, k1s = split(1.0)
  k0d, k1d = split(1.0)
  slots = NS * max(k0s + k1s, k0d + k1d)
  e_pad = slots * C
  n_pad = -(-n // (NS * C)) * (NS * C)
  if n_pad == n:
    n_pad += NS * C

  row = edge_index[0]
  col = edge_index[1]
  pad = e_pad - e
  colp = jnp.concatenate([col, jnp.zeros((pad,), jnp.int32)])
  rowp = jnp.concatenate([row, jnp.full((pad,), n, jnp.int32)])
  colm = colp.reshape(slots, C)
  rowm = rowp.reshape(slots, C)

  deg2 = _make_deg_kernel(n_pad, k0d, k1d)(rowm)

  def tc_pre(x_ref, w1_ref, b1_ref, wc_ref, bc_ref, deg_ref,
             x0_ref, taug_ref, dis_ref, dgi_ref):
    h = jnp.maximum(
        jnp.dot(x_ref[...], w1_ref[...],
                preferred_element_type=jnp.float32) + b1_ref[...], 0.0)
    x0 = jnp.dot(h, wc_ref[...], preferred_element_type=jnp.float32) \
        + bc_ref[...]
    dg = deg_ref[0, :n, :] + deg_ref[1, :n, :] + 1.0
    dis = jnp.broadcast_to(lax.rsqrt(dg), (n, out_ch))
    x0_ref[...] = x0
    dis_ref[...] = dis
    dgi_ref[...] = jnp.broadcast_to(1.0 / dg, (n, out_ch))
    taug_ref[:, 0:hid] = dis * x0
    taug_ref[:, hid:2 * hid] = dis

  x0, t0aug, dis_b, dgi_b = pl.pallas_call(
      tc_pre,
      out_shape=[
          jax.ShapeDtypeStruct((n, out_ch), jnp.float32),
          jax.ShapeDtypeStruct((n, 2 * out_ch), jnp.float32),
          jax.ShapeDtypeStruct((n, out_ch), jnp.float32),
          jax.ShapeDtypeStruct((n, out_ch), jnp.float32),
      ],
  )(x, W1, b1.reshape(1, hid), Wc, bc.reshape(1, out_ch),
    deg2.reshape(NC, n_pad, 1))

  accA = _make_spmm_kernel(n, n_pad, k0s, k1s, 2 * out_ch)(colm, rowm, t0aug)

  def tc_mid(acc_ref, x0_ref, dis_ref, dgi_ref,
             out1_ref, t1_ref, alpha_ref):
    dis = dis_ref[...]
    dgi = dgi_ref[...]
    x0v = x0_ref[...]
    s = acc_ref[0, :n, out_ch:2 * out_ch] + acc_ref[1, :n, out_ch:2 * out_ch]
    denom = dis * s + dgi + lam
    alpha = 1.0 / denom
    beta = lam * alpha
    agg = dis * (acc_ref[0, :n, 0:out_ch] + acc_ref[1, :n, 0:out_ch]) \
        + dgi * x0v
    out1 = alpha * agg + beta * x0v
    out1_ref[...] = out1
    t1_ref[...] = dis * out1
    alpha_ref[...] = alpha

  out1, t1, alpha_b = pl.pallas_call(
      tc_mid,
      out_shape=[
          jax.ShapeDtypeStruct((n, out_ch), jnp.float32),
          jax.ShapeDtypeStruct((n, out_ch), jnp.float32),
          jax.ShapeDtypeStruct((n, out_ch), jnp.float32),
      ],
  )(accA, x0, dis_b, dgi_b)

  accB = _make_spmm_kernel(n, n_pad, k0s, k1s, out_ch)(colm, rowm, t1)

  def tc_fin(acc_ref, out1_ref, alpha_ref, x0_ref, dis_ref, dgi_ref, res_ref):
    dis = dis_ref[...]
    alpha = alpha_ref[...]
    agg = dis * (acc_ref[0, :n, :] + acc_ref[1, :n, :]) \
        + dgi_ref[...] * out1_ref[...]
    out2 = alpha * agg + (lam * alpha) * x0_ref[...]
    m = jnp.max(out2, axis=1, keepdims=True)
    lse = m + jnp.log(jnp.sum(jnp.exp(out2 - m), axis=1, keepdims=True))
    res_ref[...] = out2 - lse

  res = pl.pallas_call(
      tc_fin,
      out_shape=jax.ShapeDtypeStruct((n, out_ch), jnp.float32),
  )(accB, out1, alpha_b, x0, dis_b, dgi_b)

  del k_iters
  return res


# speedup vs baseline: 1.5356x; 1.0070x over previous
import functools

import jax
import jax.numpy as jnp
from jax import lax
from jax.experimental import pallas as pl
from jax.experimental.pallas import tpu as pltpu
from jax.experimental.pallas import tpu_sc as plsc

NC = 2
NS = 16
NW = NC * NS
LANES = 16
C = 128
G = 8

_MESH = plsc.VectorSubcoreMesh(core_axis_name="c", subcore_axis_name="s")


def _zero_rows(buf, width):
  zero16 = jnp.zeros((LANES,), jnp.float32)

  def body(i, carry):
    for w0 in range(width // LANES):
      buf[i, pl.ds(w0 * LANES, LANES)] = zero16
    return carry

  lax.fori_loop(0, C, body, 0)


def _worker_range(cid, sid, k0, k1):
  my_k = jnp.where(cid == 0, k0, k1)
  start = jnp.where(cid == 0, sid * k0, NS * k0 + sid * k1)
  return start, my_k


def _make_deg_kernel(n_pad, k0, k1):
  rows_per_sub = n_pad // NS
  nz = rows_per_sub // C
  kmax = max(k0, k1)

  @functools.partial(
      pl.kernel,
      out_type=jax.ShapeDtypeStruct((NC, n_pad, LANES), jnp.float32),
      mesh=_MESH,
      compiler_params=pltpu.CompilerParams(use_tc_tiling_on_sc=False,
                                           needs_layout_passes=False),
      scratch_types=[
          pltpu.VMEM((kmax, C), jnp.int32),
          pltpu.VMEM((C,), jnp.float32),
          pltpu.VMEM((rows_per_sub,), jnp.float32),
          pltpu.VMEM((rows_per_sub + LANES,), jnp.float32),
          pltpu.VMEM((rows_per_sub, LANES), jnp.float32),
          pltpu.VMEM_SHARED((n_pad,), jnp.float32),
          pltpu.SemaphoreType.DMA,
          pltpu.SemaphoreType.DMA,
      ],
  )
  def deg_kernel(row_hbm, out_hbm, idx_v, ones_v, zbuf_v, cnt_v, brow_v,
                 acc_s, sem, sem_z):
    cid = lax.axis_index("c")
    sid = lax.axis_index("s")
    start, _ = _worker_range(cid, sid, k0, k1)
    ngroups = jnp.where(cid == 0, k0 // G, k1 // G)
    base = sid * rows_per_sub

    @pl.when(cid == 0)
    def _():
      pltpu.async_copy(row_hbm.at[pl.ds(start, k0)],
                       idx_v.at[pl.ds(0, k0)], sem)

    @pl.when(cid != 0)
    def _():
      pltpu.async_copy(row_hbm.at[pl.ds(start, k1)],
                       idx_v.at[pl.ds(0, k1)], sem)

    one16 = jnp.full((LANES,), 1.0, jnp.float32)
    zero16 = jnp.zeros((LANES,), jnp.float32)

    def fill(i, carry):
      zbuf_v[pl.ds(i * LANES, LANES)] = zero16
      return carry

    def fill1(i, carry):
      ones_v[pl.ds(i * LANES, LANES)] = one16
      return carry

    lax.fori_loop(0, rows_per_sub // LANES, fill, 0)
    lax.fori_loop(0, C // LANES, fill1, 0)

    pltpu.async_copy(zbuf_v, acc_s.at[pl.ds(base, rows_per_sub)], sem_z)
    pltpu.make_async_copy(
        zbuf_v, acc_s.at[pl.ds(base, rows_per_sub)], sem_z).wait()

    @pl.when(cid == 0)
    def _():
      pltpu.make_async_copy(row_hbm.at[pl.ds(start, k0)],
                            idx_v.at[pl.ds(0, k0)], sem).wait()

    @pl.when(cid != 0)
    def _():
      pltpu.make_async_copy(row_hbm.at[pl.ds(start, k1)],
                            idx_v.at[pl.ds(0, k1)], sem).wait()
    plsc.subcore_barrier()

    def fire(g):
      for u in range(G):
        pltpu.async_copy(ones_v, acc_s.at[idx_v.at[g * G + u]], sem, add=True)

    def drain():
      for _ in range(G):
        pltpu.make_async_copy(ones_v, acc_s.at[idx_v.at[0]], sem).wait()

    fire(0)

    def group(g, carry):
      drain()

      @pl.when(g + 1 < ngroups)
      def _():
        fire(g + 1)
      return carry

    lax.fori_loop(0, ngroups, group, 0)
    plsc.subcore_barrier()

    sl = pl.ds(base, rows_per_sub)
    pltpu.sync_copy(acc_s.at[sl], cnt_v.at[pl.ds(0, rows_per_sub)])
    cnt_v[pl.ds(rows_per_sub, LANES)] = zero16

    def brow(r, carry):
      brow_v[r, :] = jnp.full((LANES,), cnt_v[pl.ds(r, LANES)][0],
                              jnp.float32)
      return carry

    lax.fori_loop(0, rows_per_sub, brow, 0)
    pltpu.sync_copy(brow_v, out_hbm.at[cid, sl, :])

  return deg_kernel


def _make_spmm_kernel(n, n_pad, k0, k1, width):
  rows_per_sub = n_pad // NS
  tab_per_sub = n // NS
  kmax = max(k0, k1)

  @functools.partial(
      pl.kernel,
      out_type=jax.ShapeDtypeStruct((NC, n_pad, width), jnp.float32),
      mesh=_MESH,
      compiler_params=pltpu.CompilerParams(use_tc_tiling_on_sc=False),
      scratch_types=[
          pltpu.VMEM((kmax, C), jnp.int32),
          pltpu.VMEM((kmax, C), jnp.int32),
          pltpu.VMEM((2 * G * C, width), jnp.float32),
          pltpu.VMEM_SHARED((n_pad, width), jnp.float32),
          pltpu.VMEM_SHARED((n, width), jnp.float32),
          pltpu.SemaphoreType.DMA,
          pltpu.SemaphoreType.DMA,
          pltpu.SemaphoreType.DMA,
      ],
  )
  def spmm_kernel(col_hbm, row_hbm, table_hbm, out_hbm,
                  colv, rowv, gbuf, acc_s, tab_s, sem_g, sem_s, sem_z):
    cid = lax.axis_index("c")
    sid = lax.axis_index("s")
    start, _ = _worker_range(cid, sid, k0, k1)
    ngroups = jnp.where(cid == 0, k0 // G, k1 // G)
    base = sid * rows_per_sub

    tsl = pl.ds(sid * tab_per_sub, tab_per_sub)
    pltpu.async_copy(table_hbm.at[tsl, :], tab_s.at[tsl, :], sem_z)

    @pl.when(cid == 0)
    def _():
      pltpu.async_copy(col_hbm.at[pl.ds(start, k0)],
                       colv.at[pl.ds(0, k0)], sem_g)
      pltpu.async_copy(row_hbm.at[pl.ds(start, k0)],
                       rowv.at[pl.ds(0, k0)], sem_s)

    @pl.when(cid != 0)
    def _():
      pltpu.async_copy(col_hbm.at[pl.ds(start, k1)],
                       colv.at[pl.ds(0, k1)], sem_g)
      pltpu.async_copy(row_hbm.at[pl.ds(start, k1)],
                       rowv.at[pl.ds(0, k1)], sem_s)

    zero16 = jnp.zeros((LANES,), jnp.float32)

    def zfill(i, carry):
      for w0 in range(width // LANES):
        gbuf[i, pl.ds(w0 * LANES, LANES)] = zero16
      return carry

    lax.fori_loop(0, rows_per_sub, zfill, 0)
    pltpu.async_copy(gbuf.at[pl.ds(0, rows_per_sub)],
                     acc_s.at[pl.ds(base, rows_per_sub), :], sem_z)
    pltpu.make_async_copy(gbuf.at[pl.ds(0, rows_per_sub)],
                          acc_s.at[pl.ds(base, rows_per_sub), :], sem_z).wait()
    pltpu.make_async_copy(table_hbm.at[tsl, :], tab_s.at[tsl, :], sem_z).wait()

    @pl.when(cid == 0)
    def _():
      pltpu.make_async_copy(col_hbm.at[pl.ds(start, k0)],
                            colv.at[pl.ds(0, k0)], sem_g).wait()
      pltpu.make_async_copy(row_hbm.at[pl.ds(start, k0)],
                            rowv.at[pl.ds(0, k0)], sem_s).wait()

    @pl.when(cid != 0)
    def _():
      pltpu.make_async_copy(col_hbm.at[pl.ds(start, k1)],
                            colv.at[pl.ds(0, k1)], sem_g).wait()
      pltpu.make_async_copy(row_hbm.at[pl.ds(start, k1)],
                            rowv.at[pl.ds(0, k1)], sem_s).wait()
    plsc.subcore_barrier()

    def buf(slot, u):
      return gbuf.at[pl.ds((slot * G + u) * C, C), :]

    def fire_gathers(g, slot):
      for u in range(G):
        pltpu.async_copy(tab_s.at[colv.at[g * G + u]], buf(slot, u),
                         sem_g)

    def drain(sem, dst_slot):
      for u in range(G):
        pltpu.make_async_copy(tab_s.at[colv.at[0]], buf(dst_slot, u),
                              sem).wait()

    fire_gathers(0, 0)

    def group(g, carry):
      cur = lax.rem(g, 2)
      nxt = 1 - cur

      @pl.when(g >= 1)
      def _():
        drain(sem_s, nxt)
      drain(sem_g, cur)

      for u in range(G):
        pltpu.async_copy(buf(cur, u), acc_s.at[rowv.at[g * G + u]],
                         sem_s, add=True)

      @pl.when(g + 1 < ngroups)
      def _():
        fire_gathers(g + 1, nxt)
      return carry

    lax.fori_loop(0, ngroups, group, 0)
    drain(sem_s, lax.rem(ngroups - 1, 2))
    plsc.subcore_barrier()

    sl = pl.ds(base, rows_per_sub)
    pltpu.sync_copy(acc_s.at[sl, :], out_hbm.at[cid, sl, :])

  return spmm_kernel


def kernel(x, edge_index, W1, b1, Wc, bc):
  n, in_ch = x.shape
  hid = W1.shape[1]
  out_ch = Wc.shape[1]
  e = edge_index.shape[1]
  mu, p_exp, k_iters = 0.1, 2.0, 2
  lam = 2.0 * mu / p_exp
  del p_exp

  total_chunks = -(-e // C)
  pair = -(-total_chunks // NS)

  def split(r):
    kk0 = int(round(pair * r / (1.0 + r) / G)) * G
    kk0 = max(G, min(kk0, (pair // G) * G))
    kk1 = -(-(pair - kk0) // G) * G
    return kk0, kk1

  k0s, k1s = split(1.0)
  k0d, k1d = split(1.0)
  slots = NS * max(k0s + k1s, k0d + k1d)
  e_pad = slots * C
  n_pad = -(-n // (NS * C)) * (NS * C)
  if n_pad == n:
    n_pad += NS * C

  row = edge_index[0]
  col = edge_index[1]
  pad = e_pad - e
  colp = jnp.concatenate([col, jnp.zeros((pad,), jnp.int32)])
  rowp = jnp.concatenate([row, jnp.full((pad,), n, jnp.int32)])
  colm = colp.reshape(slots, C)
  rowm = rowp.reshape(slots, C)

  deg2 = _make_deg_kernel(n_pad, k0d, k1d)(rowm)

  def tc_pre(x_ref, w1_ref, b1_ref, wc_ref, bc_ref, deg_ref,
             x0_ref, taug_ref, dis_ref, dgi_ref):
    h = jnp.maximum(
        jnp.dot(x_ref[...], w1_ref[...],
                preferred_element_type=jnp.float32) + b1_ref[...], 0.0)
    x0 = jnp.dot(h, wc_ref[...], preferred_element_type=jnp.float32) \
        + bc_ref[...]
    dg = deg_ref[0, :n, :] + deg_ref[1, :n, :] + 1.0
    dis = lax.rsqrt(dg)
    x0_ref[...] = x0
    dis_ref[...] = dis
    dgi_ref[...] = 1.0 / dg
    taug_ref[:, 0:hid] = dis * x0
    taug_ref[:, hid:2 * hid] = dis

  x0, t0aug, dis_b, dgi_b = pl.pallas_call(
      tc_pre,
      out_shape=[
          jax.ShapeDtypeStruct((n, out_ch), jnp.float32),
          jax.ShapeDtypeStruct((n, 2 * out_ch), jnp.float32),
          jax.ShapeDtypeStruct((n, out_ch), jnp.float32),
          jax.ShapeDtypeStruct((n, out_ch), jnp.float32),
      ],
  )(x, W1, b1.reshape(1, hid), Wc, bc.reshape(1, out_ch), deg2)

  accA = _make_spmm_kernel(n, n_pad, k0s, k1s, 2 * out_ch)(colm, rowm, t0aug)

  def tc_mid(acc_ref, x0_ref, dis_ref, dgi_ref,
             out1_ref, t1_ref, alpha_ref):
    dis = dis_ref[...]
    dgi = dgi_ref[...]
    x0v = x0_ref[...]
    s = acc_ref[0, :n, out_ch:2 * out_ch] + acc_ref[1, :n, out_ch:2 * out_ch]
    denom = dis * s + dgi + lam
    alpha = 1.0 / denom
    beta = lam * alpha
    agg = dis * (acc_ref[0, :n, 0:out_ch] + acc_ref[1, :n, 0:out_ch]) \
        + dgi * x0v
    out1 = alpha * agg + beta * x0v
    out1_ref[...] = out1
    t1_ref[...] = dis * out1
    alpha_ref[...] = alpha

  out1, t1, alpha_b = pl.pallas_call(
      tc_mid,
      out_shape=[
          jax.ShapeDtypeStruct((n, out_ch), jnp.float32),
          jax.ShapeDtypeStruct((n, out_ch), jnp.float32),
          jax.ShapeDtypeStruct((n, out_ch), jnp.float32),
      ],
  )(accA, x0, dis_b, dgi_b)

  accB = _make_spmm_kernel(n, n_pad, k0s, k1s, out_ch)(colm, rowm, t1)

  def tc_fin(acc_ref, out1_ref, alpha_ref, x0_ref, dis_ref, dgi_ref, res_ref):
    dis = dis_ref[...]
    alpha = alpha_ref[...]
    agg = dis * (acc_ref[0, :n, :] + acc_ref[1, :n, :]) \
        + dgi_ref[...] * out1_ref[...]
    out2 = alpha * agg + (lam * alpha) * x0_ref[...]
    m = jnp.max(out2, axis=1, keepdims=True)
    lse = m + jnp.log(jnp.sum(jnp.exp(out2 - m), axis=1, keepdims=True))
    res_ref[...] = out2 - lse

  res = pl.pallas_call(
      tc_fin,
      out_shape=jax.ShapeDtypeStruct((n, out_ch), jnp.float32),
  )(accB, out1, alpha_b, x0, dis_b, dgi_b)

  del k_iters
  return res


# speedup vs baseline: 1.5428x; 1.0047x over previous
import functools

import jax
import jax.numpy as jnp
from jax import lax
from jax.experimental import pallas as pl
from jax.experimental.pallas import tpu as pltpu
from jax.experimental.pallas import tpu_sc as plsc

NC = 2
NS = 16
NW = NC * NS
LANES = 16
C = 128
G = 8

_MESH = plsc.VectorSubcoreMesh(core_axis_name="c", subcore_axis_name="s")


def _zero_rows(buf, width):
  zero16 = jnp.zeros((LANES,), jnp.float32)

  def body(i, carry):
    for w0 in range(width // LANES):
      buf[i, pl.ds(w0 * LANES, LANES)] = zero16
    return carry

  lax.fori_loop(0, C, body, 0)


def _worker_range(cid, sid, k0, k1):
  my_k = jnp.where(cid == 0, k0, k1)
  start = jnp.where(cid == 0, sid * k0, NS * k0 + sid * k1)
  return start, my_k


def _make_deg_kernel(n_pad, k0, k1):
  rows_per_sub = n_pad // NS
  nz = rows_per_sub // C
  kmax = max(k0, k1)

  @functools.partial(
      pl.kernel,
      out_type=jax.ShapeDtypeStruct((NC, n_pad, LANES), jnp.float32),
      mesh=_MESH,
      compiler_params=pltpu.CompilerParams(use_tc_tiling_on_sc=False,
                                           needs_layout_passes=False),
      scratch_types=[
          pltpu.VMEM((kmax, C), jnp.int32),
          pltpu.VMEM((C,), jnp.float32),
          pltpu.VMEM((rows_per_sub,), jnp.float32),
          pltpu.VMEM((rows_per_sub + LANES,), jnp.float32),
          pltpu.VMEM((rows_per_sub, LANES), jnp.float32),
          pltpu.VMEM_SHARED((n_pad,), jnp.float32),
          pltpu.SemaphoreType.DMA,
          pltpu.SemaphoreType.DMA,
      ],
  )
  def deg_kernel(row_hbm, out_hbm, idx_v, ones_v, zbuf_v, cnt_v, brow_v,
                 acc_s, sem, sem_z):
    cid = lax.axis_index("c")
    sid = lax.axis_index("s")
    start, _ = _worker_range(cid, sid, k0, k1)
    ngroups = jnp.where(cid == 0, k0 // G, k1 // G)
    base = sid * rows_per_sub

    @pl.when(cid == 0)
    def _():
      pltpu.async_copy(row_hbm.at[pl.ds(start, k0)],
                       idx_v.at[pl.ds(0, k0)], sem)

    @pl.when(cid != 0)
    def _():
      pltpu.async_copy(row_hbm.at[pl.ds(start, k1)],
                       idx_v.at[pl.ds(0, k1)], sem)

    one16 = jnp.full((LANES,), 1.0, jnp.float32)
    zero16 = jnp.zeros((LANES,), jnp.float32)

    def fill(i, carry):
      zbuf_v[pl.ds(i * LANES, LANES)] = zero16
      return carry

    def fill1(i, carry):
      ones_v[pl.ds(i * LANES, LANES)] = one16
      return carry

    lax.fori_loop(0, rows_per_sub // LANES, fill, 0)
    lax.fori_loop(0, C // LANES, fill1, 0)

    pltpu.async_copy(zbuf_v, acc_s.at[pl.ds(base, rows_per_sub)], sem_z)
    pltpu.make_async_copy(
        zbuf_v, acc_s.at[pl.ds(base, rows_per_sub)], sem_z).wait()

    @pl.when(cid == 0)
    def _():
      pltpu.make_async_copy(row_hbm.at[pl.ds(start, k0)],
                            idx_v.at[pl.ds(0, k0)], sem).wait()

    @pl.when(cid != 0)
    def _():
      pltpu.make_async_copy(row_hbm.at[pl.ds(start, k1)],
                            idx_v.at[pl.ds(0, k1)], sem).wait()
    plsc.subcore_barrier()

    def fire(g):
      for u in range(G):
        pltpu.async_copy(ones_v, acc_s.at[idx_v.at[g * G + u]], sem, add=True)

    def drain():
      for _ in range(G):
        pltpu.make_async_copy(ones_v, acc_s.at[idx_v.at[0]], sem).wait()

    fire(0)

    def group(g, carry):
      drain()

      @pl.when(g + 1 < ngroups)
      def _():
        fire(g + 1)
      return carry

    lax.fori_loop(0, ngroups, group, 0)
    plsc.subcore_barrier()

    sl = pl.ds(base, rows_per_sub)
    pltpu.sync_copy(acc_s.at[sl], cnt_v.at[pl.ds(0, rows_per_sub)])
    cnt_v[pl.ds(rows_per_sub, LANES)] = zero16

    def brow(r, carry):
      brow_v[r, :] = jnp.full((LANES,), cnt_v[pl.ds(r, LANES)][0],
                              jnp.float32)
      return carry

    lax.fori_loop(0, rows_per_sub, brow, 0)
    pltpu.sync_copy(brow_v, out_hbm.at[cid, sl, :])

  return deg_kernel


def _make_spmm_kernel(n, n_pad, k0, k1, width):
  rows_per_sub = n_pad // NS
  tab_per_sub = n // NS
  kmax = max(k0, k1)

  @functools.partial(
      pl.kernel,
      out_type=jax.ShapeDtypeStruct((NC, n_pad, width), jnp.float32),
      mesh=_MESH,
      compiler_params=pltpu.CompilerParams(use_tc_tiling_on_sc=False),
      scratch_types=[
          pltpu.VMEM((kmax, C), jnp.int32),
          pltpu.VMEM((kmax, C), jnp.int32),
          pltpu.VMEM((2 * G * C, width), jnp.float32),
          pltpu.VMEM_SHARED((n_pad, width), jnp.float32),
          pltpu.VMEM_SHARED((n, width), jnp.float32),
          pltpu.SemaphoreType.DMA,
          pltpu.SemaphoreType.DMA,
          pltpu.SemaphoreType.DMA,
      ],
  )
  def spmm_kernel(col_hbm, row_hbm, table_hbm, out_hbm,
                  colv, rowv, gbuf, acc_s, tab_s, sem_g, sem_s, sem_z):
    cid = lax.axis_index("c")
    sid = lax.axis_index("s")
    start, _ = _worker_range(cid, sid, k0, k1)
    ngroups = jnp.where(cid == 0, k0 // G, k1 // G)
    base = sid * rows_per_sub

    tsl = pl.ds(sid * tab_per_sub, tab_per_sub)
    pltpu.async_copy(table_hbm.at[tsl, :], tab_s.at[tsl, :], sem_z)

    @pl.when(cid == 0)
    def _():
      pltpu.async_copy(col_hbm.at[pl.ds(start, k0)],
                       colv.at[pl.ds(0, k0)], sem_g)
      pltpu.async_copy(row_hbm.at[pl.ds(start, k0)],
                       rowv.at[pl.ds(0, k0)], sem_s)

    @pl.when(cid != 0)
    def _():
      pltpu.async_copy(col_hbm.at[pl.ds(start, k1)],
                       colv.at[pl.ds(0, k1)], sem_g)
      pltpu.async_copy(row_hbm.at[pl.ds(start, k1)],
                       rowv.at[pl.ds(0, k1)], sem_s)

    zero16 = jnp.zeros((LANES,), jnp.float32)

    def zfill(i, carry):
      for w0 in range(width // LANES):
        gbuf[i, pl.ds(w0 * LANES, LANES)] = zero16
      return carry

    lax.fori_loop(0, rows_per_sub, zfill, 0)
    pltpu.async_copy(gbuf.at[pl.ds(0, rows_per_sub)],
                     acc_s.at[pl.ds(base, rows_per_sub), :], sem_z)
    pltpu.make_async_copy(gbuf.at[pl.ds(0, rows_per_sub)],
                          acc_s.at[pl.ds(base, rows_per_sub), :], sem_z).wait()
    pltpu.make_async_copy(table_hbm.at[tsl, :], tab_s.at[tsl, :], sem_z).wait()

    @pl.when(cid == 0)
    def _():
      pltpu.make_async_copy(col_hbm.at[pl.ds(start, k0)],
                            colv.at[pl.ds(0, k0)], sem_g).wait()
      pltpu.make_async_copy(row_hbm.at[pl.ds(start, k0)],
                            rowv.at[pl.ds(0, k0)], sem_s).wait()

    @pl.when(cid != 0)
    def _():
      pltpu.make_async_copy(col_hbm.at[pl.ds(start, k1)],
                            colv.at[pl.ds(0, k1)], sem_g).wait()
      pltpu.make_async_copy(row_hbm.at[pl.ds(start, k1)],
                            rowv.at[pl.ds(0, k1)], sem_s).wait()
    plsc.subcore_barrier()

    def buf(slot, u):
      return gbuf.at[pl.ds((slot * G + u) * C, C), :]

    def fire_gathers(g, slot):
      for u in range(G):
        pltpu.async_copy(tab_s.at[colv.at[g * G + u]], buf(slot, u),
                         sem_g)

    def drain(sem, dst_slot):
      for u in range(G):
        pltpu.make_async_copy(tab_s.at[colv.at[0]], buf(dst_slot, u),
                              sem).wait()

    fire_gathers(0, 0)

    def group(g, carry):
      cur = lax.rem(g, 2)
      nxt = 1 - cur

      @pl.when(g >= 1)
      def _():
        drain(sem_s, nxt)
      drain(sem_g, cur)

      for u in range(G):
        pltpu.async_copy(buf(cur, u), acc_s.at[rowv.at[g * G + u]],
                         sem_s, add=True)

      @pl.when(g + 1 < ngroups)
      def _():
        fire_gathers(g + 1, nxt)
      return carry

    lax.fori_loop(0, ngroups, group, 0)
    drain(sem_s, lax.rem(ngroups - 1, 2))
    plsc.subcore_barrier()

    sl = pl.ds(base, rows_per_sub)
    pltpu.sync_copy(acc_s.at[sl, :], out_hbm.at[cid, sl, :])

  return spmm_kernel


def kernel(x, edge_index, W1, b1, Wc, bc):
  n, in_ch = x.shape
  hid = W1.shape[1]
  out_ch = Wc.shape[1]
  e = edge_index.shape[1]
  mu, p_exp, k_iters = 0.1, 2.0, 2
  lam = 2.0 * mu / p_exp
  del p_exp

  total_chunks = -(-e // C)
  pair = -(-total_chunks // NS)

  def split(r):
    kk0 = int(round(pair * r / (1.0 + r) / G)) * G
    kk0 = max(G, min(kk0, (pair // G) * G))
    kk1 = -(-(pair - kk0) // G) * G
    return kk0, kk1

  k0s, k1s = split(1.15)
  k0d, k1d = split(1.8)
  slots = NS * max(k0s + k1s, k0d + k1d)
  e_pad = slots * C
  n_pad = -(-n // (NS * C)) * (NS * C)
  if n_pad == n:
    n_pad += NS * C

  row = edge_index[0]
  col = edge_index[1]
  pad = e_pad - e
  colp = jnp.concatenate([col, jnp.zeros((pad,), jnp.int32)])
  rowp = jnp.concatenate([row, jnp.full((pad,), n, jnp.int32)])
  colm = colp.reshape(slots, C)
  rowm = rowp.reshape(slots, C)

  deg2 = _make_deg_kernel(n_pad, k0d, k1d)(rowm)

  def tc_pre(x_ref, w1_ref, b1_ref, wc_ref, bc_ref, deg_ref,
             x0_ref, taug_ref, dis_ref, dgi_ref):
    h = jnp.maximum(
        jnp.dot(x_ref[...], w1_ref[...],
                preferred_element_type=jnp.float32) + b1_ref[...], 0.0)
    x0 = jnp.dot(h, wc_ref[...], preferred_element_type=jnp.float32) \
        + bc_ref[...]
    dg = deg_ref[0, :n, :] + deg_ref[1, :n, :] + 1.0
    dis = lax.rsqrt(dg)
    x0_ref[...] = x0
    dis_ref[...] = dis
    dgi_ref[...] = 1.0 / dg
    taug_ref[:, 0:hid] = dis * x0
    taug_ref[:, hid:2 * hid] = dis

  x0, t0aug, dis_b, dgi_b = pl.pallas_call(
      tc_pre,
      out_shape=[
          jax.ShapeDtypeStruct((n, out_ch), jnp.float32),
          jax.ShapeDtypeStruct((n, 2 * out_ch), jnp.float32),
          jax.ShapeDtypeStruct((n, out_ch), jnp.float32),
          jax.ShapeDtypeStruct((n, out_ch), jnp.float32),
      ],
  )(x, W1, b1.reshape(1, hid), Wc, bc.reshape(1, out_ch), deg2)

  accA = _make_spmm_kernel(n, n_pad, k0s, k1s, 2 * out_ch)(colm, rowm, t0aug)

  def tc_mid(acc_ref, x0_ref, dis_ref, dgi_ref,
             out1_ref, t1_ref, alpha_ref):
    dis = dis_ref[...]
    dgi = dgi_ref[...]
    x0v = x0_ref[...]
    s = acc_ref[0, :n, out_ch:2 * out_ch] + acc_ref[1, :n, out_ch:2 * out_ch]
    denom = dis * s + dgi + lam
    alpha = 1.0 / denom
    beta = lam * alpha
    agg = dis * (acc_ref[0, :n, 0:out_ch] + acc_ref[1, :n, 0:out_ch]) \
        + dgi * x0v
    out1 = alpha * agg + beta * x0v
    out1_ref[...] = out1
    t1_ref[...] = dis * out1
    alpha_ref[...] = alpha

  out1, t1, alpha_b = pl.pallas_call(
      tc_mid,
      out_shape=[
          jax.ShapeDtypeStruct((n, out_ch), jnp.float32),
          jax.ShapeDtypeStruct((n, out_ch), jnp.float32),
          jax.ShapeDtypeStruct((n, out_ch), jnp.float32),
      ],
  )(accA, x0, dis_b, dgi_b)

  accB = _make_spmm_kernel(n, n_pad, k0s, k1s, out_ch)(colm, rowm, t1)

  def tc_fin(acc_ref, out1_ref, alpha_ref, x0_ref, dis_ref, dgi_ref, res_ref):
    dis = dis_ref[...]
    alpha = alpha_ref[...]
    agg = dis * (acc_ref[0, :n, :] + acc_ref[1, :n, :]) \
        + dgi_ref[...] * out1_ref[...]
    out2 = alpha * agg + (lam * alpha) * x0_ref[...]
    m = jnp.max(out2, axis=1, keepdims=True)
    lse = m + jnp.log(jnp.sum(jnp.exp(out2 - m), axis=1, keepdims=True))
    res_ref[...] = out2 - lse

  res = pl.pallas_call(
      tc_fin,
      out_shape=jax.ShapeDtypeStruct((n, out_ch), jnp.float32),
  )(accB, out1, alpha_b, x0, dis_b, dgi_b)

  del k_iters
  return res
